# baseline (speedup 1.0000x reference)
import functools
import math

import jax
import jax.numpy as jnp
from jax import lax
from jax.experimental import pallas as pl
from jax.experimental.pallas import tpu as pltpu
from jax.experimental.pallas import tpu_sc as plsc

NN = 10000
NE = 160000
NG = 556
NP = 10240
EP = 163840
GP = 640
D = 128
H = 4
HD = 32
NL = 6
CUTOFF = 5.0
AVG_DEGREE = 15.57930850982666
AVG_NUM_NODES = 18.03065905448718

NB_E = 2048
NB_N = 1024

_SC_NC = 2
_SC_NS = 16
_SC_NW = _SC_NC * _SC_NS


def _sc_gather(table, idx, chunk, out_w=None):
    R = idx.shape[0]
    Nt = table.shape[0]
    Dw = table.shape[1]
    out_w = Dw if out_w is None else out_w
    per_w = R // _SC_NW
    n_chunks = per_w // chunk
    t_stripe = Nt // _SC_NS
    mesh = plsc.VectorSubcoreMesh(core_axis_name="c", subcore_axis_name="s")

    depth = 2
    nslot = 4

    @functools.partial(
        pl.kernel, mesh=mesh,
        out_type=jax.ShapeDtypeStruct((R, out_w), table.dtype),
        scratch_types=(
            [pltpu.VMEM((per_w,), jnp.int32)]
            + [pltpu.VMEM((chunk, Dw), table.dtype)] * nslot
            + [pltpu.VMEM_SHARED((Nt, Dw), table.dtype)]
            + [pltpu.SemaphoreType.DMA] * (2 * nslot)
        ),
    )
    def k(table_hbm, idx_hbm, out_hbm, idx_v, *bufs_sems):
        bufs = bufs_sems[:nslot]
        table_sh = bufs_sems[nslot]
        sg = bufs_sems[nslot + 1:2 * nslot + 1]
        sw = bufs_sems[2 * nslot + 1:]
        cid = lax.axis_index("c")
        sid = lax.axis_index("s")
        wid = sid * _SC_NC + cid
        base = wid * per_w
        pltpu.sync_copy(table_hbm.at[pl.ds(sid * t_stripe, t_stripe), :],
                        table_sh.at[pl.ds(sid * t_stripe, t_stripe), :])
        pltpu.sync_copy(idx_hbm.at[pl.ds(base, per_w)], idx_v)
        plsc.subcore_barrier()

        def g_start(c):
            b = c % nslot
            return pltpu.async_copy(
                table_sh.at[idx_v.at[pl.ds(c * chunk, chunk)]], bufs[b], sg[b])

        def wb_start(c):
            b = c % nslot
            return pltpu.async_copy(
                bufs[b].at[:, pl.ds(0, out_w)],
                out_hbm.at[pl.ds(base + c * chunk, chunk), :], sw[b])

        hg = {c: g_start(c) for c in range(min(depth, n_chunks))}
        hw = {}
        for c in range(n_chunks):
            hg.pop(c).wait()
            hw[c] = wb_start(c)
            nxt = c + depth
            if nxt < n_chunks:
                old = nxt - nslot
                if old >= 0:
                    hw.pop(old).wait()
                hg[nxt] = g_start(nxt)
        for h in hw.values():
            h.wait()

    return k(table, idx)


def _sc_scatter_add(rows, idx3, zeros, n_out, chunk, group):
    R, W = rows.shape
    per_core = R // _SC_NC
    per_w = per_core // _SC_NS
    n_chunks = per_w // chunk
    n_groups = n_chunks // group
    stripe = n_out // _SC_NS
    mesh = plsc.VectorSubcoreMesh(core_axis_name="c", subcore_axis_name="s")

    @functools.partial(
        pl.kernel, mesh=mesh,
        out_type=jax.ShapeDtypeStruct((_SC_NC, n_out, W), rows.dtype),
        scratch_types=(
            [pltpu.VMEM((n_chunks, 1, chunk), jnp.int32)]
            + [pltpu.VMEM((chunk, W), rows.dtype)] * _SCAT_NSLOT
            + [pltpu.VMEM_SHARED((n_out, W), rows.dtype)]
            + [pltpu.SemaphoreType.DMA] * (2 * _SCAT_NSLOT)
        ),
    )
    def k(rows_hbm, idx_hbm, zeros_hbm, out_hbm, idx_v, *rest):
        bufs = rest[:_SCAT_NSLOT]
        acc_sh = rest[_SCAT_NSLOT]
        scp = rest[_SCAT_NSLOT + 1:2 * _SCAT_NSLOT + 1]
        sad = rest[2 * _SCAT_NSLOT + 1:]
        cid = lax.axis_index("c")
        sid = lax.axis_index("s")
        base = cid * per_core + sid * per_w
        pltpu.sync_copy(idx_hbm.at[pl.ds(base // chunk, n_chunks), :, :], idx_v)
        pltpu.sync_copy(zeros_hbm.at[pl.ds(sid * stripe, stripe), :],
                        acc_sh.at[pl.ds(sid * stripe, stripe), :])
        plsc.subcore_barrier()

        def cp(c, buf, sem):
            return pltpu.async_copy(
                rows_hbm.at[pl.ds(base + c * chunk, chunk), :], buf, sem)

        def add(c, buf, sem):
            return pltpu.async_copy(buf, acc_sh.at[idx_v.at[c, 0]], sem, add=True)

        _pipe_scatter(cp, add, n_chunks, bufs, scp, sad)
        plsc.subcore_barrier()
        pltpu.sync_copy(acc_sh.at[pl.ds(sid * stripe, stripe), :],
                        out_hbm.at[cid, pl.ds(sid * stripe, stripe), :])

    return k(rows, idx3, zeros)


_SCAT_DEPTH = 1
_SCAT_NSLOT = 3


def _pipe_scatter(cp, add, n_chunks, bufs, scp, sad):
    nslot = len(bufs)
    hc = {c: cp(c, bufs[c % nslot], scp[c % nslot])
          for c in range(min(_SCAT_DEPTH, n_chunks))}
    ha = {}
    for c in range(n_chunks):
        hc.pop(c).wait()
        ha[c] = add(c, bufs[c % nslot], sad[c % nslot])
        nxt = c + _SCAT_DEPTH
        if nxt < n_chunks:
            old = nxt - nslot
            if old >= 0:
                ha.pop(old).wait()
            hc[nxt] = cp(nxt, bufs[nxt % nslot], scp[nxt % nslot])
    for h in ha.values():
        h.wait()


def _sc_scatter_add2(rows_a, rows_b, idx3, zeros, n_out, chunk, group):
    R, W = rows_a.shape
    per_w = R // _SC_NS
    n_chunks = per_w // chunk
    n_groups = n_chunks // group
    stripe = n_out // _SC_NS
    mesh = plsc.VectorSubcoreMesh(core_axis_name="c", subcore_axis_name="s")

    @functools.partial(
        pl.kernel, mesh=mesh,
        out_type=[jax.ShapeDtypeStruct((n_out, W), rows_a.dtype),
                  jax.ShapeDtypeStruct((n_out, W), rows_b.dtype)],
        scratch_types=(
            [pltpu.VMEM((n_chunks, 1, chunk), jnp.int32)]
            + [pltpu.VMEM((chunk, W), rows_a.dtype)] * _SCAT_NSLOT
            + [pltpu.VMEM_SHARED((n_out, W), rows_a.dtype)]
            + [pltpu.SemaphoreType.DMA] * (2 * _SCAT_NSLOT)
        ),
    )
    def k(ra_hbm, rb_hbm, idx_hbm, zeros_hbm, oa_hbm, ob_hbm, idx_v, *rest):
        bufs = rest[:_SCAT_NSLOT]
        acc_sh = rest[_SCAT_NSLOT]
        scp = rest[_SCAT_NSLOT + 1:2 * _SCAT_NSLOT + 1]
        sad = rest[2 * _SCAT_NSLOT + 1:]
        cid = lax.axis_index("c")
        sid = lax.axis_index("s")
        base = sid * per_w
        pltpu.sync_copy(idx_hbm.at[pl.ds(base // chunk, n_chunks), :, :], idx_v)
        pltpu.sync_copy(zeros_hbm.at[pl.ds(sid * stripe, stripe), :],
                        acc_sh.at[pl.ds(sid * stripe, stripe), :])
        plsc.subcore_barrier()

        def add(c, buf, sem):
            return pltpu.async_copy(buf, acc_sh.at[idx_v.at[c, 0]], sem, add=True)

        def run(rows_hbm):
            def cp(c, buf, sem):
                return pltpu.async_copy(
                    rows_hbm.at[pl.ds(base + c * chunk, chunk), :], buf, sem)
            _pipe_scatter(cp, add, n_chunks, bufs, scp, sad)

        @pl.when(cid == 0)
        def _():
            run(ra_hbm)

        @pl.when(cid == 1)
        def _():
            run(rb_hbm)

        plsc.subcore_barrier()

        @pl.when(cid == 0)
        def _():
            pltpu.sync_copy(acc_sh.at[pl.ds(sid * stripe, stripe), :],
                            oa_hbm.at[pl.ds(sid * stripe, stripe), :])

        @pl.when(cid == 1)
        def _():
            pltpu.sync_copy(acc_sh.at[pl.ds(sid * stripe, stripe), :],
                            ob_hbm.at[pl.ds(sid * stripe, stripe), :])

    return k(rows_a, rows_b, idx3, zeros)


def _silu(x):
    return x / (1.0 + jnp.exp(-x))


def _ln_rows(x, g, b):
    mu = jnp.mean(x, axis=-1, keepdims=True)
    xc = x - mu
    var = jnp.mean(xc * xc, axis=-1, keepdims=True)
    return g * xc * lax.rsqrt(var + 1e-5) + b


def _head_expand():
    col = lax.broadcasted_iota(jnp.int32, (H, D), 1) // HD
    row = lax.broadcasted_iota(jnp.int32, (H, D), 0)
    return (col == row).astype(jnp.float32)


def _edge_init_body(ps_ref, pd_ref, means_ref, w1_ref, b1_ref, w2_ref, b2_ref,
                    w3_ref, rbf_ref, ew_ref):
    pid = pl.program_id(0)
    lane = lax.broadcasted_iota(jnp.int32, (NB_E, D), 1)
    dvec = jnp.where(lane < 3, pd_ref[...] - ps_ref[...], 0.0)
    d2 = jnp.sum(dvec * dvec, axis=-1, keepdims=True)
    dist = jnp.sqrt(d2 + 1e-12)
    cut = 0.5 * (jnp.cos(dist * (math.pi / CUTOFF)) + 1.0)
    cut = jnp.where(dist < CUTOFF, cut, 0.0)
    start = math.exp(-CUTOFF)
    beta = (2.0 / 128.0 * (1.0 - start)) ** (-2)
    t = jnp.exp(-dist) - means_ref[...]
    rbf = cut * jnp.exp(-beta * t * t)
    rbf_ref[...] = rbf
    h = _silu(jnp.dot(rbf, w1_ref[...], preferred_element_type=jnp.float32) + b1_ref[...])
    h = _silu(jnp.dot(h, w2_ref[...], preferred_element_type=jnp.float32) + b2_ref[...])
    ew = jnp.dot(h, w3_ref[...], preferred_element_type=jnp.float32)
    eid = pid * NB_E + lax.broadcasted_iota(jnp.int32, (NB_E, 1), 0)
    emask = (eid < NE).astype(jnp.float32)
    ew_ref[...] = ew * emask


def _combine0_body(emb_ref, p0_ref, p1_ref, g_ref, b_ref, x_ref, xn_ref):
    seg = p0_ref[...] + p1_ref[...]
    x = emb_ref[...] + seg * (1.0 / math.sqrt(AVG_DEGREE))
    x_ref[...] = x
    xn_ref[...] = _ln_rows(x, g_ref[...], b_ref[...])


def _edge_attn_body(rbf_ref, xs_ref, xd_ref, wq_ref, wk_ref, wv_ref,
                    wep_ref, msg_ref, e_ref):
    pid = pl.program_id(0)
    q = jnp.dot(xd_ref[...], wq_ref[...], preferred_element_type=jnp.float32)
    kin = xs_ref[...] + jnp.dot(rbf_ref[...], wep_ref[...],
                                preferred_element_type=jnp.float32)
    k = jnp.dot(kin, wk_ref[...], preferred_element_type=jnp.float32)
    v = jnp.dot(kin, wv_ref[...], preferred_element_type=jnp.float32)
    e4 = _head_expand()
    prod = q * k
    logits = jnp.dot(prod, e4.T, preferred_element_type=jnp.float32) * (1.0 / math.sqrt(HD))
    logits = jnp.clip(logits, -75.0, 75.0)
    eid = pid * NB_E + lax.broadcasted_iota(jnp.int32, (NB_E, 1), 0)
    emask = (eid < NE).astype(jnp.float32)
    e = jnp.exp(logits) * emask
    ebc = jnp.dot(e, e4, preferred_element_type=jnp.float32)
    msg_ref[...] = v * ebc
    e_ref[...] = ebc


def _node_update_body(x_ref, p_ref, q_ref, wo_ref, wf1_ref,
                      wf2_ref, g2_ref, b2_ref, gn_ref, bn_ref, xo_ref, xn_ref):
    u = p_ref[...]
    zr = q_ref[...] + 1e-9
    agg = u / zr
    x = x_ref[...] + jnp.dot(agg, wo_ref[...], preferred_element_type=jnp.float32)
    xn2 = _ln_rows(x, g2_ref[...], b2_ref[...])
    f = _silu(jnp.dot(xn2, wf1_ref[...], preferred_element_type=jnp.float32))
    xo = x + jnp.dot(f, wf2_ref[...], preferred_element_type=jnp.float32)
    xo_ref[...] = xo
    xn_ref[...] = _ln_rows(xo, gn_ref[...], bn_ref[...])


def _head_body(xf_ref, w1_ref, b1_ref, w2_ref, b2_ref, o_ref):
    pid = pl.program_id(0)
    xf = xf_ref[...]
    hh = _silu(jnp.dot(xf, w1_ref[...], preferred_element_type=jnp.float32) + b1_ref[...])
    o = jnp.dot(hh, w2_ref[...], preferred_element_type=jnp.float32) + b2_ref[...]
    nid = pid * NB_N + lax.broadcasted_iota(jnp.int32, (NB_N, 1), 0)
    nmask = (nid < NN).astype(jnp.float32)
    o_ref[...] = o * nmask


def _full(shape):
    return pl.BlockSpec(shape, lambda i: tuple(0 for _ in shape))


def _tc_edge_init(ps, pd, means, w1, b1, w2, b2, w3):
    eb = pl.BlockSpec((NB_E, D), lambda i: (i, 0))
    eb16 = pl.BlockSpec((NB_E, 16), lambda i: (i, 0))
    return pl.pallas_call(
        _edge_init_body,
        grid=(EP // NB_E,),
        in_specs=[eb, eb, _full((1, D)), _full((D, 64)), _full((1, 64)),
                  _full((64, 64)), _full((1, 64)), _full((64, D))],
        out_specs=[eb, eb],
        out_shape=[jax.ShapeDtypeStruct((EP, D), jnp.float32),
                   jax.ShapeDtypeStruct((EP, D), jnp.float32)],
    )(ps, pd, means, w1, b1, w2, b2, w3)


def _tc_combine0(emb, p0, p1, g, b):
    nb = pl.BlockSpec((NB_N, D), lambda i: (i, 0))
    return pl.pallas_call(
        _combine0_body,
        grid=(NP // NB_N,),
        in_specs=[nb, nb, nb, _full((1, D)), _full((1, D))],
        out_specs=[nb, nb],
        out_shape=[jax.ShapeDtypeStruct((NP, D), jnp.float32)] * 2,
    )(emb, p0, p1, g, b)


def _tc_edge_attn(rbf, xs, xd, wq, wk, wv, wep):
    eb = pl.BlockSpec((NB_E, D), lambda i: (i, 0))
    wb = _full((D, D))
    return pl.pallas_call(
        _edge_attn_body,
        grid=(EP // NB_E,),
        in_specs=[eb, eb, eb, wb, wb, wb, wb],
        out_specs=[eb, eb],
        out_shape=[jax.ShapeDtypeStruct((EP, D), jnp.float32),
                   jax.ShapeDtypeStruct((EP, D), jnp.float32)],
    )(rbf, xs, xd, wq, wk, wv, wep)


def _tc_node_update(x, p, q, wo, wf1, wf2, g2, b2, gn, bn):
    nb = pl.BlockSpec((NB_N, D), lambda i: (i, 0))
    return pl.pallas_call(
        _node_update_body,
        grid=(NP // NB_N,),
        in_specs=[nb, nb, nb, _full((D, D)), _full((D, 2 * D)),
                  _full((2 * D, D)), _full((1, D)), _full((1, D)),
                  _full((1, D)), _full((1, D))],
        out_specs=[nb, nb],
        out_shape=[jax.ShapeDtypeStruct((NP, D), jnp.float32)] * 2,
    )(x, p, q, wo, wf1, wf2, g2, b2, gn, bn)


def _tc_head(xf, w1, b1, w2, b2):
    nb = pl.BlockSpec((NB_N, D), lambda i: (i, 0))
    return pl.pallas_call(
        _head_body,
        grid=(NP // NB_N,),
        in_specs=[nb, _full((D, 512)),
                  _full((1, 512)), _full((512, D)), _full((1, D))],
        out_specs=nb,
        out_shape=jax.ShapeDtypeStruct((NP, D), jnp.float32),
    )(xf, w1, b1, w2, b2)


def kernel(node_atom, pos, batch, edge_index, atom_table, deg_w1, deg_b1,
           deg_w2, deg_b2, deg_w3, Wq, Wk, Wv, Wo, Weproj, Wf1, Wf2,
           ln1_g, ln1_b, ln2_g, ln2_b, lnf_g, lnf_b,
           head_w1, head_b1, head_w2, head_b2):
    f32 = jnp.float32
    src = jnp.pad(edge_index[0].astype(jnp.int32), (0, EP - NE))
    dst = jnp.pad(edge_index[1].astype(jnp.int32), (0, EP - NE))
    cat_idx = jnp.concatenate([src, dst])
    atom_p = jnp.pad(node_atom.astype(jnp.int32), (0, NP - NN))
    batch_p = jnp.pad(batch.astype(jnp.int32), (0, NP - NN),
                      constant_values=NG)
    pos_p = jnp.pad(pos.astype(f32), ((0, NP - NN), (0, D - 3)))

    start = math.exp(-CUTOFF)
    means = (start + (1.0 - start) / 127.0
             * jnp.arange(128, dtype=f32)).reshape(1, D)
    z_n = jnp.zeros((NP, D), f32)
    z_g = jnp.zeros((GP, D), f32)
    w2p = jnp.pad(head_w2.astype(f32), ((0, 0), (0, D - 1)))
    b2p = jnp.pad(head_b2.astype(f32), (0, D - 1)).reshape(1, D)

    dst3 = dst.reshape(EP // 64, 1, 64)
    batch3 = batch_p.reshape(NP // 80, 1, 80)

    posg = _sc_gather(pos_p, cat_idx, 64)
    rbf, ew = _tc_edge_init(posg[:EP], posg[EP:], means, deg_w1,
                            deg_b1.reshape(1, 64), deg_w2,
                            deg_b2.reshape(1, 64), deg_w3)
    pew = _sc_scatter_add(ew, dst3, z_n, NP, 64, 8)
    emb = _sc_gather(atom_table.astype(f32), atom_p, 80)
    x, xn = _tc_combine0(emb, pew[0], pew[1],
                         ln1_g[0].reshape(1, D), ln1_b[0].reshape(1, D))

    for i in range(NL):
        xg = _sc_gather(xn, cat_idx, 64)
        msg, e128 = _tc_edge_attn(rbf, xg[:EP], xg[EP:], Wq[i], Wk[i], Wv[i],
                                  Weproj[i])
        p, q = _sc_scatter_add2(msg, e128, dst3, z_n, NP, 64, 8)
        if i < NL - 1:
            gn, bn = ln1_g[i + 1], ln1_b[i + 1]
        else:
            gn, bn = lnf_g, lnf_b
        x, xn = _tc_node_update(x, p, q, Wo[i], Wf1[i], Wf2[i],
                                ln2_g[i].reshape(1, D), ln2_b[i].reshape(1, D),
                                gn.reshape(1, D), bn.reshape(1, D))

    no128 = _tc_head(xn, head_w1, head_b1.reshape(1, 512), w2p, b2p)
    pg = _sc_scatter_add(no128, batch3, z_g, GP, 80, 4)
    out = (pg[0, :NG, 0:1] + pg[1, :NG, 0:1]) * (1.0 / math.sqrt(AVG_NUM_NODES))
    return out


# speedup vs baseline: 1.0784x; 1.0784x over previous
import functools
import math

import jax
import jax.numpy as jnp
from jax import lax
from jax.experimental import pallas as pl
from jax.experimental.pallas import tpu as pltpu
from jax.experimental.pallas import tpu_sc as plsc

NN = 10000
NE = 160000
NG = 556
NP = 10240
EP = 163840
GP = 640
D = 128
H = 4
HD = 32
NL = 6
CUTOFF = 5.0
AVG_DEGREE = 15.57930850982666
AVG_NUM_NODES = 18.03065905448718

NB_E = 2048
NB_N = 1024

_SC_NC = 2
_SC_NS = 16
_SC_NW = _SC_NC * _SC_NS


def _sc_gather(table, idx, chunk, out_w=None):
    R = idx.shape[0]
    Nt = table.shape[0]
    Dw = table.shape[1]
    out_w = Dw if out_w is None else out_w
    per_w = R // _SC_NW
    n_chunks = per_w // chunk
    t_stripe = Nt // _SC_NS
    mesh = plsc.VectorSubcoreMesh(core_axis_name="c", subcore_axis_name="s")

    depth = 1
    nslot = 2

    @functools.partial(
        pl.kernel, mesh=mesh,
        out_type=jax.ShapeDtypeStruct((R, out_w), table.dtype),
        scratch_types=(
            [pltpu.VMEM((per_w,), jnp.int32)]
            + [pltpu.VMEM((chunk, Dw), table.dtype)] * nslot
            + [pltpu.VMEM_SHARED((Nt, Dw), table.dtype)]
            + [pltpu.SemaphoreType.DMA] * (2 * nslot)
        ),
    )
    def k(table_hbm, idx_hbm, out_hbm, idx_v, *bufs_sems):
        bufs = bufs_sems[:nslot]
        table_sh = bufs_sems[nslot]
        sg = bufs_sems[nslot + 1:2 * nslot + 1]
        sw = bufs_sems[2 * nslot + 1:]
        cid = lax.axis_index("c")
        sid = lax.axis_index("s")
        wid = sid * _SC_NC + cid
        base = wid * per_w
        pltpu.sync_copy(table_hbm.at[pl.ds(sid * t_stripe, t_stripe), :],
                        table_sh.at[pl.ds(sid * t_stripe, t_stripe), :])
        pltpu.sync_copy(idx_hbm.at[pl.ds(base, per_w)], idx_v)
        plsc.subcore_barrier()

        def g_start(c):
            b = c % nslot
            return pltpu.async_copy(
                table_sh.at[idx_v.at[pl.ds(c * chunk, chunk)]], bufs[b], sg[b])

        def wb_start(c):
            b = c % nslot
            return pltpu.async_copy(
                bufs[b].at[:, pl.ds(0, out_w)],
                out_hbm.at[pl.ds(base + c * chunk, chunk), :], sw[b])

        hg = {c: g_start(c) for c in range(min(depth, n_chunks))}
        hw = {}
        for c in range(n_chunks):
            hg.pop(c).wait()
            hw[c] = wb_start(c)
            nxt = c + depth
            if nxt < n_chunks:
                old = nxt - nslot
                if old >= 0:
                    hw.pop(old).wait()
                hg[nxt] = g_start(nxt)
        for h in hw.values():
            h.wait()

    return k(table, idx)


def _sc_scatter_add(rows, idx3, zeros, n_out, chunk, group):
    R, W = rows.shape
    per_core = R // _SC_NC
    per_w = per_core // _SC_NS
    n_chunks = per_w // chunk
    n_groups = n_chunks // group
    stripe = n_out // _SC_NS
    mesh = plsc.VectorSubcoreMesh(core_axis_name="c", subcore_axis_name="s")

    @functools.partial(
        pl.kernel, mesh=mesh,
        out_type=jax.ShapeDtypeStruct((_SC_NC, n_out, W), rows.dtype),
        scratch_types=(
            [pltpu.VMEM((n_chunks, 1, chunk), jnp.int32)]
            + [pltpu.VMEM((chunk, W), rows.dtype)] * _SCAT_NSLOT
            + [pltpu.VMEM_SHARED((n_out, W), rows.dtype)]
            + [pltpu.SemaphoreType.DMA] * (2 * _SCAT_NSLOT)
        ),
    )
    def k(rows_hbm, idx_hbm, zeros_hbm, out_hbm, idx_v, *rest):
        bufs = rest[:_SCAT_NSLOT]
        acc_sh = rest[_SCAT_NSLOT]
        scp = rest[_SCAT_NSLOT + 1:2 * _SCAT_NSLOT + 1]
        sad = rest[2 * _SCAT_NSLOT + 1:]
        cid = lax.axis_index("c")
        sid = lax.axis_index("s")
        base = cid * per_core + sid * per_w
        pltpu.sync_copy(idx_hbm.at[pl.ds(base // chunk, n_chunks), :, :], idx_v)
        pltpu.sync_copy(zeros_hbm.at[pl.ds(sid * stripe, stripe), :],
                        acc_sh.at[pl.ds(sid * stripe, stripe), :])
        plsc.subcore_barrier()

        def cp(c, buf, sem):
            return pltpu.async_copy(
                rows_hbm.at[pl.ds(base + c * chunk, chunk), :], buf, sem)

        def add(c, buf, sem):
            return pltpu.async_copy(buf, acc_sh.at[idx_v.at[c, 0]], sem, add=True)

        _pipe_scatter(cp, add, n_chunks, bufs, scp, sad)
        plsc.subcore_barrier()
        pltpu.sync_copy(acc_sh.at[pl.ds(sid * stripe, stripe), :],
                        out_hbm.at[cid, pl.ds(sid * stripe, stripe), :])

    return k(rows, idx3, zeros)


_SCAT_DEPTH = 1
_SCAT_NSLOT = 2


def _pipe_scatter(cp, add, n_chunks, bufs, scp, sad):
    nslot = len(bufs)
    hc = {c: cp(c, bufs[c % nslot], scp[c % nslot])
          for c in range(min(_SCAT_DEPTH, n_chunks))}
    ha = {}
    for c in range(n_chunks):
        hc.pop(c).wait()
        ha[c] = add(c, bufs[c % nslot], sad[c % nslot])
        nxt = c + _SCAT_DEPTH
        if nxt < n_chunks:
            old = nxt - nslot
            if old >= 0:
                ha.pop(old).wait()
            hc[nxt] = cp(nxt, bufs[nxt % nslot], scp[nxt % nslot])
    for h in ha.values():
        h.wait()


def _sc_scatter_add2(rows_a, rows_b, idx3, zeros, n_out, chunk, group):
    R, W = rows_a.shape
    per_w = R // _SC_NS
    n_chunks = per_w // chunk
    n_groups = n_chunks // group
    stripe = n_out // _SC_NS
    mesh = plsc.VectorSubcoreMesh(core_axis_name="c", subcore_axis_name="s")

    @functools.partial(
        pl.kernel, mesh=mesh,
        out_type=[jax.ShapeDtypeStruct((n_out, W), rows_a.dtype),
                  jax.ShapeDtypeStruct((n_out, W), rows_b.dtype)],
        scratch_types=(
            [pltpu.VMEM((n_chunks, 1, chunk), jnp.int32)]
            + [pltpu.VMEM((chunk, W), rows_a.dtype)] * _SCAT_NSLOT
            + [pltpu.VMEM_SHARED((n_out, W), rows_a.dtype)]
            + [pltpu.SemaphoreType.DMA] * (2 * _SCAT_NSLOT)
        ),
    )
    def k(ra_hbm, rb_hbm, idx_hbm, zeros_hbm, oa_hbm, ob_hbm, idx_v, *rest):
        bufs = rest[:_SCAT_NSLOT]
        acc_sh = rest[_SCAT_NSLOT]
        scp = rest[_SCAT_NSLOT + 1:2 * _SCAT_NSLOT + 1]
        sad = rest[2 * _SCAT_NSLOT + 1:]
        cid = lax.axis_index("c")
        sid = lax.axis_index("s")
        base = sid * per_w
        pltpu.sync_copy(idx_hbm.at[pl.ds(base // chunk, n_chunks), :, :], idx_v)
        pltpu.sync_copy(zeros_hbm.at[pl.ds(sid * stripe, stripe), :],
                        acc_sh.at[pl.ds(sid * stripe, stripe), :])
        plsc.subcore_barrier()

        def add(c, buf, sem):
            return pltpu.async_copy(buf, acc_sh.at[idx_v.at[c, 0]], sem, add=True)

        def run(rows_hbm):
            def cp(c, buf, sem):
                return pltpu.async_copy(
                    rows_hbm.at[pl.ds(base + c * chunk, chunk), :], buf, sem)
            _pipe_scatter(cp, add, n_chunks, bufs, scp, sad)

        @pl.when(cid == 0)
        def _():
            run(ra_hbm)

        @pl.when(cid == 1)
        def _():
            run(rb_hbm)

        plsc.subcore_barrier()

        @pl.when(cid == 0)
        def _():
            pltpu.sync_copy(acc_sh.at[pl.ds(sid * stripe, stripe), :],
                            oa_hbm.at[pl.ds(sid * stripe, stripe), :])

        @pl.when(cid == 1)
        def _():
            pltpu.sync_copy(acc_sh.at[pl.ds(sid * stripe, stripe), :],
                            ob_hbm.at[pl.ds(sid * stripe, stripe), :])

    return k(rows_a, rows_b, idx3, zeros)


def _silu(x):
    return x / (1.0 + jnp.exp(-x))


def _ln_rows(x, g, b):
    mu = jnp.mean(x, axis=-1, keepdims=True)
    xc = x - mu
    var = jnp.mean(xc * xc, axis=-1, keepdims=True)
    return g * xc * lax.rsqrt(var + 1e-5) + b


def _head_expand():
    col = lax.broadcasted_iota(jnp.int32, (H, D), 1) // HD
    row = lax.broadcasted_iota(jnp.int32, (H, D), 0)
    return (col == row).astype(jnp.float32)


def _edge_init_body(ps_ref, pd_ref, means_ref, w1_ref, b1_ref, w2_ref, b2_ref,
                    w3_ref, rbf_ref, ew_ref):
    pid = pl.program_id(0)
    lane = lax.broadcasted_iota(jnp.int32, (NB_E, D), 1)
    dvec = jnp.where(lane < 3, pd_ref[...] - ps_ref[...], 0.0)
    d2 = jnp.sum(dvec * dvec, axis=-1, keepdims=True)
    dist = jnp.sqrt(d2 + 1e-12)
    cut = 0.5 * (jnp.cos(dist * (math.pi / CUTOFF)) + 1.0)
    cut = jnp.where(dist < CUTOFF, cut, 0.0)
    start = math.exp(-CUTOFF)
    beta = (2.0 / 128.0 * (1.0 - start)) ** (-2)
    t = jnp.exp(-dist) - means_ref[...]
    rbf = cut * jnp.exp(-beta * t * t)
    rbf_ref[...] = rbf
    h = _silu(jnp.dot(rbf, w1_ref[...], preferred_element_type=jnp.float32) + b1_ref[...])
    h = _silu(jnp.dot(h, w2_ref[...], preferred_element_type=jnp.float32) + b2_ref[...])
    ew = jnp.dot(h, w3_ref[...], preferred_element_type=jnp.float32)
    eid = pid * NB_E + lax.broadcasted_iota(jnp.int32, (NB_E, 1), 0)
    emask = (eid < NE).astype(jnp.float32)
    ew_ref[...] = ew * emask


def _combine0_body(emb_ref, p0_ref, p1_ref, g_ref, b_ref, x_ref, xn_ref):
    seg = p0_ref[...] + p1_ref[...]
    x = emb_ref[...] + seg * (1.0 / math.sqrt(AVG_DEGREE))
    x_ref[...] = x
    xn_ref[...] = _ln_rows(x, g_ref[...], b_ref[...])


def _edge_attn_body(rbf_ref, xs_ref, xd_ref, wq_ref, wk_ref, wv_ref,
                    wep_ref, msg_ref, e_ref):
    pid = pl.program_id(0)
    q = jnp.dot(xd_ref[...], wq_ref[...], preferred_element_type=jnp.float32)
    kin = xs_ref[...] + jnp.dot(rbf_ref[...], wep_ref[...],
                                preferred_element_type=jnp.float32)
    k = jnp.dot(kin, wk_ref[...], preferred_element_type=jnp.float32)
    v = jnp.dot(kin, wv_ref[...], preferred_element_type=jnp.float32)
    e4 = _head_expand()
    prod = q * k
    logits = jnp.dot(prod, e4.T, preferred_element_type=jnp.float32) * (1.0 / math.sqrt(HD))
    logits = jnp.clip(logits, -75.0, 75.0)
    eid = pid * NB_E + lax.broadcasted_iota(jnp.int32, (NB_E, 1), 0)
    emask = (eid < NE).astype(jnp.float32)
    e = jnp.exp(logits) * emask
    ebc = jnp.dot(e, e4, preferred_element_type=jnp.float32)
    msg_ref[...] = v * ebc
    e_ref[...] = ebc


def _node_update_body(x_ref, p_ref, q_ref, wo_ref, wf1_ref,
                      wf2_ref, g2_ref, b2_ref, gn_ref, bn_ref, xo_ref, xn_ref):
    u = p_ref[...]
    zr = q_ref[...] + 1e-9
    agg = u / zr
    x = x_ref[...] + jnp.dot(agg, wo_ref[...], preferred_element_type=jnp.float32)
    xn2 = _ln_rows(x, g2_ref[...], b2_ref[...])
    f = _silu(jnp.dot(xn2, wf1_ref[...], preferred_element_type=jnp.float32))
    xo = x + jnp.dot(f, wf2_ref[...], preferred_element_type=jnp.float32)
    xo_ref[...] = xo
    xn_ref[...] = _ln_rows(xo, gn_ref[...], bn_ref[...])


def _head_body(xf_ref, w1_ref, b1_ref, w2_ref, b2_ref, o_ref):
    pid = pl.program_id(0)
    xf = xf_ref[...]
    hh = _silu(jnp.dot(xf, w1_ref[...], preferred_element_type=jnp.float32) + b1_ref[...])
    o = jnp.dot(hh, w2_ref[...], preferred_element_type=jnp.float32) + b2_ref[...]
    nid = pid * NB_N + lax.broadcasted_iota(jnp.int32, (NB_N, 1), 0)
    nmask = (nid < NN).astype(jnp.float32)
    o_ref[...] = o * nmask


def _full(shape):
    return pl.BlockSpec(shape, lambda i: tuple(0 for _ in shape))


def _tc_edge_init(ps, pd, means, w1, b1, w2, b2, w3):
    eb = pl.BlockSpec((NB_E, D), lambda i: (i, 0))
    eb16 = pl.BlockSpec((NB_E, 16), lambda i: (i, 0))
    return pl.pallas_call(
        _edge_init_body,
        grid=(EP // NB_E,),
        in_specs=[eb, eb, _full((1, D)), _full((D, 64)), _full((1, 64)),
                  _full((64, 64)), _full((1, 64)), _full((64, D))],
        out_specs=[eb, eb],
        out_shape=[jax.ShapeDtypeStruct((EP, D), jnp.float32),
                   jax.ShapeDtypeStruct((EP, D), jnp.float32)],
    )(ps, pd, means, w1, b1, w2, b2, w3)


def _tc_combine0(emb, p0, p1, g, b):
    nb = pl.BlockSpec((NB_N, D), lambda i: (i, 0))
    return pl.pallas_call(
        _combine0_body,
        grid=(NP // NB_N,),
        in_specs=[nb, nb, nb, _full((1, D)), _full((1, D))],
        out_specs=[nb, nb],
        out_shape=[jax.ShapeDtypeStruct((NP, D), jnp.float32)] * 2,
    )(emb, p0, p1, g, b)


def _tc_edge_attn(rbf, xs, xd, wq, wk, wv, wep):
    eb = pl.BlockSpec((NB_E, D), lambda i: (i, 0))
    wb = _full((D, D))
    return pl.pallas_call(
        _edge_attn_body,
        grid=(EP // NB_E,),
        in_specs=[eb, eb, eb, wb, wb, wb, wb],
        out_specs=[eb, eb],
        out_shape=[jax.ShapeDtypeStruct((EP, D), jnp.float32),
                   jax.ShapeDtypeStruct((EP, D), jnp.float32)],
    )(rbf, xs, xd, wq, wk, wv, wep)


def _tc_node_update(x, p, q, wo, wf1, wf2, g2, b2, gn, bn):
    nb = pl.BlockSpec((NB_N, D), lambda i: (i, 0))
    return pl.pallas_call(
        _node_update_body,
        grid=(NP // NB_N,),
        in_specs=[nb, nb, nb, _full((D, D)), _full((D, 2 * D)),
                  _full((2 * D, D)), _full((1, D)), _full((1, D)),
                  _full((1, D)), _full((1, D))],
        out_specs=[nb, nb],
        out_shape=[jax.ShapeDtypeStruct((NP, D), jnp.float32)] * 2,
    )(x, p, q, wo, wf1, wf2, g2, b2, gn, bn)


def _tc_head(xf, w1, b1, w2, b2):
    nb = pl.BlockSpec((NB_N, D), lambda i: (i, 0))
    return pl.pallas_call(
        _head_body,
        grid=(NP // NB_N,),
        in_specs=[nb, _full((D, 512)),
                  _full((1, 512)), _full((512, D)), _full((1, D))],
        out_specs=nb,
        out_shape=jax.ShapeDtypeStruct((NP, D), jnp.float32),
    )(xf, w1, b1, w2, b2)


def kernel(node_atom, pos, batch, edge_index, atom_table, deg_w1, deg_b1,
           deg_w2, deg_b2, deg_w3, Wq, Wk, Wv, Wo, Weproj, Wf1, Wf2,
           ln1_g, ln1_b, ln2_g, ln2_b, lnf_g, lnf_b,
           head_w1, head_b1, head_w2, head_b2):
    f32 = jnp.float32
    src = jnp.pad(edge_index[0].astype(jnp.int32), (0, EP - NE))
    dst = jnp.pad(edge_index[1].astype(jnp.int32), (0, EP - NE))
    cat_idx = jnp.concatenate([src, dst])
    atom_p = jnp.pad(node_atom.astype(jnp.int32), (0, NP - NN))
    batch_p = jnp.pad(batch.astype(jnp.int32), (0, NP - NN),
                      constant_values=NG)
    pos_p = jnp.pad(pos.astype(f32), ((0, NP - NN), (0, D - 3)))

    start = math.exp(-CUTOFF)
    means = (start + (1.0 - start) / 127.0
             * jnp.arange(128, dtype=f32)).reshape(1, D)
    z_n = jnp.zeros((NP, D), f32)
    z_g = jnp.zeros((GP, D), f32)
    w2p = jnp.pad(head_w2.astype(f32), ((0, 0), (0, D - 1)))
    b2p = jnp.pad(head_b2.astype(f32), (0, D - 1)).reshape(1, D)

    dst3 = dst.reshape(EP // 128, 1, 128)
    batch3 = batch_p.reshape(NP // 80, 1, 80)

    posg = _sc_gather(pos_p, cat_idx, 128)
    rbf, ew = _tc_edge_init(posg[:EP], posg[EP:], means, deg_w1,
                            deg_b1.reshape(1, 64), deg_w2,
                            deg_b2.reshape(1, 64), deg_w3)
    pew = _sc_scatter_add(ew, dst3, z_n, NP, 128, 8)
    emb = _sc_gather(atom_table.astype(f32), atom_p, 80)
    x, xn = _tc_combine0(emb, pew[0], pew[1],
                         ln1_g[0].reshape(1, D), ln1_b[0].reshape(1, D))

    for i in range(NL):
        xg = _sc_gather(xn, cat_idx, 128)
        msg, e128 = _tc_edge_attn(rbf, xg[:EP], xg[EP:], Wq[i], Wk[i], Wv[i],
                                  Weproj[i])
        p, q = _sc_scatter_add2(msg, e128, dst3, z_n, NP, 128, 8)
        if i < NL - 1:
            gn, bn = ln1_g[i + 1], ln1_b[i + 1]
        else:
            gn, bn = lnf_g, lnf_b
        x, xn = _tc_node_update(x, p, q, Wo[i], Wf1[i], Wf2[i],
                                ln2_g[i].reshape(1, D), ln2_b[i].reshape(1, D),
                                gn.reshape(1, D), bn.reshape(1, D))

    no128 = _tc_head(xn, head_w1, head_b1.reshape(1, 512), w2p, b2p)
    pg = _sc_scatter_add(no128, batch3, z_g, GP, 80, 4)
    out = (pg[0, :NG, 0:1] + pg[1, :NG, 0:1]) * (1.0 / math.sqrt(AVG_NUM_NODES))
    return out


# speedup vs baseline: 1.1158x; 1.0347x over previous
import functools
import math

import jax
import jax.numpy as jnp
from jax import lax
from jax.experimental import pallas as pl
from jax.experimental.pallas import tpu as pltpu
from jax.experimental.pallas import tpu_sc as plsc

NN = 10000
NE = 160000
NG = 556
NP = 10240
EP = 163840
GP = 640
D = 128
H = 4
HD = 32
NL = 6
CUTOFF = 5.0
AVG_DEGREE = 15.57930850982666
AVG_NUM_NODES = 18.03065905448718

NB_E = 2048
NB_N = 1024

_SC_NC = 2
_SC_NS = 16
_SC_NW = _SC_NC * _SC_NS


def _sc_gather(table, idx, chunk, out_w=None):
    R = idx.shape[0]
    Nt = table.shape[0]
    Dw = table.shape[1]
    out_w = Dw if out_w is None else out_w
    per_w = R // _SC_NW
    n_chunks = per_w // chunk
    t_stripe = Nt // _SC_NS
    mesh = plsc.VectorSubcoreMesh(core_axis_name="c", subcore_axis_name="s")

    depth = 1
    nslot = 2

    @functools.partial(
        pl.kernel, mesh=mesh,
        out_type=jax.ShapeDtypeStruct((R, out_w), table.dtype),
        scratch_types=(
            [pltpu.VMEM((per_w,), jnp.int32)]
            + [pltpu.VMEM((chunk, Dw), table.dtype)] * nslot
            + [pltpu.VMEM_SHARED((Nt, Dw), table.dtype)]
            + [pltpu.SemaphoreType.DMA] * (2 * nslot)
        ),
    )
    def k(table_hbm, idx_hbm, out_hbm, idx_v, *bufs_sems):
        bufs = bufs_sems[:nslot]
        table_sh = bufs_sems[nslot]
        sg = bufs_sems[nslot + 1:2 * nslot + 1]
        sw = bufs_sems[2 * nslot + 1:]
        cid = lax.axis_index("c")
        sid = lax.axis_index("s")
        wid = sid * _SC_NC + cid
        base = wid * per_w
        pltpu.sync_copy(table_hbm.at[pl.ds(sid * t_stripe, t_stripe), :],
                        table_sh.at[pl.ds(sid * t_stripe, t_stripe), :])
        pltpu.sync_copy(idx_hbm.at[pl.ds(base, per_w)], idx_v)
        plsc.subcore_barrier()

        def g_start(c):
            b = c % nslot
            return pltpu.async_copy(
                table_sh.at[idx_v.at[pl.ds(c * chunk, chunk)]], bufs[b], sg[b])

        def wb_start(c):
            b = c % nslot
            return pltpu.async_copy(
                bufs[b].at[:, pl.ds(0, out_w)],
                out_hbm.at[pl.ds(base + c * chunk, chunk), :], sw[b])

        hg = {c: g_start(c) for c in range(min(depth, n_chunks))}
        hw = {}
        for c in range(n_chunks):
            hg.pop(c).wait()
            hw[c] = wb_start(c)
            nxt = c + depth
            if nxt < n_chunks:
                old = nxt - nslot
                if old >= 0:
                    hw.pop(old).wait()
                hg[nxt] = g_start(nxt)
        for h in hw.values():
            h.wait()

    return k(table, idx)


def _sc_scatter_add(rows, idx3, zeros, n_out, chunk, group):
    R, W = rows.shape
    per_core = R // _SC_NC
    per_w = per_core // _SC_NS
    n_chunks = per_w // chunk
    n_groups = n_chunks // group
    stripe = n_out // _SC_NS
    mesh = plsc.VectorSubcoreMesh(core_axis_name="c", subcore_axis_name="s")

    @functools.partial(
        pl.kernel, mesh=mesh,
        out_type=jax.ShapeDtypeStruct((_SC_NC, n_out, W), rows.dtype),
        scratch_types=(
            [pltpu.VMEM((n_chunks, 1, chunk), jnp.int32)]
            + [pltpu.VMEM((chunk, W), rows.dtype)] * _SCAT_NSLOT
            + [pltpu.VMEM_SHARED((n_out, W), rows.dtype)]
            + [pltpu.SemaphoreType.DMA] * (2 * _SCAT_NSLOT)
        ),
    )
    def k(rows_hbm, idx_hbm, zeros_hbm, out_hbm, idx_v, *rest):
        bufs = rest[:_SCAT_NSLOT]
        acc_sh = rest[_SCAT_NSLOT]
        scp = rest[_SCAT_NSLOT + 1:2 * _SCAT_NSLOT + 1]
        sad = rest[2 * _SCAT_NSLOT + 1:]
        cid = lax.axis_index("c")
        sid = lax.axis_index("s")
        base = cid * per_core + sid * per_w
        pltpu.sync_copy(idx_hbm.at[pl.ds(base // chunk, n_chunks), :, :], idx_v)
        pltpu.sync_copy(zeros_hbm.at[pl.ds(sid * stripe, stripe), :],
                        acc_sh.at[pl.ds(sid * stripe, stripe), :])
        plsc.subcore_barrier()

        def cp(c, buf, sem):
            return pltpu.async_copy(
                rows_hbm.at[pl.ds(base + c * chunk, chunk), :], buf, sem)

        def add(c, buf, sem):
            return pltpu.async_copy(buf, acc_sh.at[idx_v.at[c, 0]], sem, add=True)

        _pipe_scatter(cp, add, n_chunks, bufs, scp, sad)
        plsc.subcore_barrier()
        pltpu.sync_copy(acc_sh.at[pl.ds(sid * stripe, stripe), :],
                        out_hbm.at[cid, pl.ds(sid * stripe, stripe), :])

    return k(rows, idx3, zeros)


_SCAT_DEPTH = 1
_SCAT_NSLOT = 2


def _pipe_scatter(cp, add, n_chunks, bufs, scp, sad):
    nslot = len(bufs)
    hc = {c: cp(c, bufs[c % nslot], scp[c % nslot])
          for c in range(min(_SCAT_DEPTH, n_chunks))}
    ha = {}
    for c in range(n_chunks):
        hc.pop(c).wait()
        ha[c] = add(c, bufs[c % nslot], sad[c % nslot])
        nxt = c + _SCAT_DEPTH
        if nxt < n_chunks:
            old = nxt - nslot
            if old >= 0:
                ha.pop(old).wait()
            hc[nxt] = cp(nxt, bufs[nxt % nslot], scp[nxt % nslot])
    for h in ha.values():
        h.wait()


def _sc_scatter_add2(rows_a, rows_b, idx3, zeros, n_out, chunk, group):
    R, W = rows_a.shape
    per_w = R // _SC_NS
    n_chunks = per_w // chunk
    n_groups = n_chunks // group
    stripe = n_out // _SC_NS
    mesh = plsc.VectorSubcoreMesh(core_axis_name="c", subcore_axis_name="s")

    @functools.partial(
        pl.kernel, mesh=mesh,
        out_type=[jax.ShapeDtypeStruct((n_out, W), rows_a.dtype),
                  jax.ShapeDtypeStruct((n_out, W), rows_b.dtype)],
        scratch_types=(
            [pltpu.VMEM((n_chunks, 1, chunk), jnp.int32)]
            + [pltpu.VMEM((chunk, W), rows_a.dtype)] * _SCAT_NSLOT
            + [pltpu.VMEM_SHARED((n_out, W), rows_a.dtype)]
            + [pltpu.SemaphoreType.DMA] * (2 * _SCAT_NSLOT)
        ),
    )
    def k(ra_hbm, rb_hbm, idx_hbm, zeros_hbm, oa_hbm, ob_hbm, idx_v, *rest):
        bufs = rest[:_SCAT_NSLOT]
        acc_sh = rest[_SCAT_NSLOT]
        scp = rest[_SCAT_NSLOT + 1:2 * _SCAT_NSLOT + 1]
        sad = rest[2 * _SCAT_NSLOT + 1:]
        cid = lax.axis_index("c")
        sid = lax.axis_index("s")
        base = sid * per_w
        pltpu.sync_copy(idx_hbm.at[pl.ds(base // chunk, n_chunks), :, :], idx_v)
        pltpu.sync_copy(zeros_hbm.at[pl.ds(sid * stripe, stripe), :],
                        acc_sh.at[pl.ds(sid * stripe, stripe), :])
        plsc.subcore_barrier()

        def add(c, buf, sem):
            return pltpu.async_copy(buf, acc_sh.at[idx_v.at[c, 0]], sem, add=True)

        def run(rows_hbm):
            def cp(c, buf, sem):
                return pltpu.async_copy(
                    rows_hbm.at[pl.ds(base + c * chunk, chunk), :], buf, sem)
            _pipe_scatter(cp, add, n_chunks, bufs, scp, sad)

        @pl.when(cid == 0)
        def _():
            run(ra_hbm)

        @pl.when(cid == 1)
        def _():
            run(rb_hbm)

        plsc.subcore_barrier()

        @pl.when(cid == 0)
        def _():
            pltpu.sync_copy(acc_sh.at[pl.ds(sid * stripe, stripe), :],
                            oa_hbm.at[pl.ds(sid * stripe, stripe), :])

        @pl.when(cid == 1)
        def _():
            pltpu.sync_copy(acc_sh.at[pl.ds(sid * stripe, stripe), :],
                            ob_hbm.at[pl.ds(sid * stripe, stripe), :])

    return k(rows_a, rows_b, idx3, zeros)


def _silu(x):
    return x / (1.0 + jnp.exp(-x))


def _ln_rows(x, g, b):
    mu = jnp.mean(x, axis=-1, keepdims=True)
    xc = x - mu
    var = jnp.mean(xc * xc, axis=-1, keepdims=True)
    return g * xc * lax.rsqrt(var + 1e-5) + b


def _head_expand():
    col = lax.broadcasted_iota(jnp.int32, (H, D), 1) // HD
    row = lax.broadcasted_iota(jnp.int32, (H, D), 0)
    return (col == row).astype(jnp.float32)


def _edge_init_body(ps_ref, pd_ref, means_ref, w1_ref, b1_ref, w2_ref, b2_ref,
                    w3_ref, rbf_ref, ew_ref):
    pid = pl.program_id(0)
    lane = lax.broadcasted_iota(jnp.int32, (NB_E, D), 1)
    dvec = jnp.where(lane < 3, pd_ref[...] - ps_ref[...], 0.0)
    d2 = jnp.sum(dvec * dvec, axis=-1, keepdims=True)
    dist = jnp.sqrt(d2 + 1e-12)
    cut = 0.5 * (jnp.cos(dist * (math.pi / CUTOFF)) + 1.0)
    cut = jnp.where(dist < CUTOFF, cut, 0.0)
    start = math.exp(-CUTOFF)
    beta = (2.0 / 128.0 * (1.0 - start)) ** (-2)
    t = jnp.exp(-dist) - means_ref[...]
    rbf = cut * jnp.exp(-beta * t * t)
    rbf_ref[...] = rbf
    h = _silu(jnp.dot(rbf, w1_ref[...], preferred_element_type=jnp.float32) + b1_ref[...])
    h = _silu(jnp.dot(h, w2_ref[...], preferred_element_type=jnp.float32) + b2_ref[...])
    ew = jnp.dot(h, w3_ref[...], preferred_element_type=jnp.float32)
    eid = pid * NB_E + lax.broadcasted_iota(jnp.int32, (NB_E, 1), 0)
    emask = (eid < NE).astype(jnp.float32)
    ew_ref[...] = ew * emask


def _combine0_body(emb_ref, p0_ref, p1_ref, g_ref, b_ref, x_ref, xn_ref):
    seg = p0_ref[...] + p1_ref[...]
    x = emb_ref[...] + seg * (1.0 / math.sqrt(AVG_DEGREE))
    x_ref[...] = x
    xn_ref[...] = _ln_rows(x, g_ref[...], b_ref[...])


def _edge_attn_body(base, rbf_ref, xs_ref, xd_ref, wq_ref, wk_ref, wv_ref,
                    wep_ref, msg_ref, e_ref):
    pid = pl.program_id(0)
    q = jnp.dot(xd_ref[...], wq_ref[...], preferred_element_type=jnp.float32)
    kin = xs_ref[...] + jnp.dot(rbf_ref[...], wep_ref[...],
                                preferred_element_type=jnp.float32)
    k = jnp.dot(kin, wk_ref[...], preferred_element_type=jnp.float32)
    v = jnp.dot(kin, wv_ref[...], preferred_element_type=jnp.float32)
    e4 = _head_expand()
    prod = q * k
    logits = jnp.dot(prod, e4.T, preferred_element_type=jnp.float32) * (1.0 / math.sqrt(HD))
    logits = jnp.clip(logits, -75.0, 75.0)
    eid = base + pid * NB_E + lax.broadcasted_iota(jnp.int32, (NB_E, 1), 0)
    emask = (eid < NE).astype(jnp.float32)
    e = jnp.exp(logits) * emask
    ebc = jnp.dot(e, e4, preferred_element_type=jnp.float32)
    msg_ref[...] = v * ebc
    e_ref[...] = ebc


def _node_update_body(x_ref, pa_ref, qa_ref, pb_ref, qb_ref, wo_ref, wf1_ref,
                      wf2_ref, g2_ref, b2_ref, gn_ref, bn_ref, xo_ref, xn_ref):
    u = pa_ref[...] + pb_ref[...]
    zr = qa_ref[...] + qb_ref[...] + 1e-9
    agg = u / zr
    x = x_ref[...] + jnp.dot(agg, wo_ref[...], preferred_element_type=jnp.float32)
    xn2 = _ln_rows(x, g2_ref[...], b2_ref[...])
    f = _silu(jnp.dot(xn2, wf1_ref[...], preferred_element_type=jnp.float32))
    xo = x + jnp.dot(f, wf2_ref[...], preferred_element_type=jnp.float32)
    xo_ref[...] = xo
    xn_ref[...] = _ln_rows(xo, gn_ref[...], bn_ref[...])


def _head_body(xf_ref, w1_ref, b1_ref, w2_ref, b2_ref, o_ref):
    pid = pl.program_id(0)
    xf = xf_ref[...]
    hh = _silu(jnp.dot(xf, w1_ref[...], preferred_element_type=jnp.float32) + b1_ref[...])
    o = jnp.dot(hh, w2_ref[...], preferred_element_type=jnp.float32) + b2_ref[...]
    nid = pid * NB_N + lax.broadcasted_iota(jnp.int32, (NB_N, 1), 0)
    nmask = (nid < NN).astype(jnp.float32)
    o_ref[...] = o * nmask


def _full(shape):
    return pl.BlockSpec(shape, lambda i: tuple(0 for _ in shape))


def _tc_edge_init(ps, pd, means, w1, b1, w2, b2, w3):
    eb = pl.BlockSpec((NB_E, D), lambda i: (i, 0))
    eb16 = pl.BlockSpec((NB_E, 16), lambda i: (i, 0))
    return pl.pallas_call(
        _edge_init_body,
        grid=(EP // NB_E,),
        in_specs=[eb, eb, _full((1, D)), _full((D, 64)), _full((1, 64)),
                  _full((64, 64)), _full((1, 64)), _full((64, D))],
        out_specs=[eb, eb],
        out_shape=[jax.ShapeDtypeStruct((EP, D), jnp.float32),
                   jax.ShapeDtypeStruct((EP, D), jnp.float32)],
    )(ps, pd, means, w1, b1, w2, b2, w3)


def _tc_combine0(emb, p0, p1, g, b):
    nb = pl.BlockSpec((NB_N, D), lambda i: (i, 0))
    return pl.pallas_call(
        _combine0_body,
        grid=(NP // NB_N,),
        in_specs=[nb, nb, nb, _full((1, D)), _full((1, D))],
        out_specs=[nb, nb],
        out_shape=[jax.ShapeDtypeStruct((NP, D), jnp.float32)] * 2,
    )(emb, p0, p1, g, b)


def _tc_edge_attn(rbf, xs, xd, wq, wk, wv, wep, base):
    ne = rbf.shape[0]
    eb = pl.BlockSpec((NB_E, D), lambda i: (i, 0))
    wb = _full((D, D))
    return pl.pallas_call(
        functools.partial(_edge_attn_body, base),
        grid=(ne // NB_E,),
        in_specs=[eb, eb, eb, wb, wb, wb, wb],
        out_specs=[eb, eb],
        out_shape=[jax.ShapeDtypeStruct((ne, D), jnp.float32),
                   jax.ShapeDtypeStruct((ne, D), jnp.float32)],
    )(rbf, xs, xd, wq, wk, wv, wep)


def _tc_node_update(x, pa, qa, pb, qb, wo, wf1, wf2, g2, b2, gn, bn):
    nb = pl.BlockSpec((NB_N, D), lambda i: (i, 0))
    return pl.pallas_call(
        _node_update_body,
        grid=(NP // NB_N,),
        in_specs=[nb, nb, nb, nb, nb, _full((D, D)), _full((D, 2 * D)),
                  _full((2 * D, D)), _full((1, D)), _full((1, D)),
                  _full((1, D)), _full((1, D))],
        out_specs=[nb, nb],
        out_shape=[jax.ShapeDtypeStruct((NP, D), jnp.float32)] * 2,
    )(x, pa, qa, pb, qb, wo, wf1, wf2, g2, b2, gn, bn)


def _tc_head(xf, w1, b1, w2, b2):
    nb = pl.BlockSpec((NB_N, D), lambda i: (i, 0))
    return pl.pallas_call(
        _head_body,
        grid=(NP // NB_N,),
        in_specs=[nb, _full((D, 512)),
                  _full((1, 512)), _full((512, D)), _full((1, D))],
        out_specs=nb,
        out_shape=jax.ShapeDtypeStruct((NP, D), jnp.float32),
    )(xf, w1, b1, w2, b2)


def kernel(node_atom, pos, batch, edge_index, atom_table, deg_w1, deg_b1,
           deg_w2, deg_b2, deg_w3, Wq, Wk, Wv, Wo, Weproj, Wf1, Wf2,
           ln1_g, ln1_b, ln2_g, ln2_b, lnf_g, lnf_b,
           head_w1, head_b1, head_w2, head_b2):
    f32 = jnp.float32
    src = jnp.pad(edge_index[0].astype(jnp.int32), (0, EP - NE))
    dst = jnp.pad(edge_index[1].astype(jnp.int32), (0, EP - NE))
    cat_idx = jnp.concatenate([src, dst])
    atom_p = jnp.pad(node_atom.astype(jnp.int32), (0, NP - NN))
    batch_p = jnp.pad(batch.astype(jnp.int32), (0, NP - NN),
                      constant_values=NG)
    pos_p = jnp.pad(pos.astype(f32), ((0, NP - NN), (0, D - 3)))

    start = math.exp(-CUTOFF)
    means = (start + (1.0 - start) / 127.0
             * jnp.arange(128, dtype=f32)).reshape(1, D)
    z_n = jnp.zeros((NP, D), f32)
    z_g = jnp.zeros((GP, D), f32)
    w2p = jnp.pad(head_w2.astype(f32), ((0, 0), (0, D - 1)))
    b2p = jnp.pad(head_b2.astype(f32), (0, D - 1)).reshape(1, D)

    dst3 = dst.reshape(EP // 128, 1, 128)
    batch3 = batch_p.reshape(NP // 80, 1, 80)

    posg = _sc_gather(pos_p, cat_idx, 128)
    rbf, ew = _tc_edge_init(posg[:EP], posg[EP:], means, deg_w1,
                            deg_b1.reshape(1, 64), deg_w2,
                            deg_b2.reshape(1, 64), deg_w3)
    pew = _sc_scatter_add(ew, dst3, z_n, NP, 128, 8)
    emb = _sc_gather(atom_table.astype(f32), atom_p, 80)
    x, xn = _tc_combine0(emb, pew[0], pew[1],
                         ln1_g[0].reshape(1, D), ln1_b[0].reshape(1, D))

    EH = EP // 2
    catA = jnp.concatenate([src[:EH], dst[:EH]])
    catB = jnp.concatenate([src[EH:], dst[EH:]])
    dstA3 = dst3[:EH // 128]
    dstB3 = dst3[EH // 128:]

    for i in range(NL):
        wq, wk, wv, wep = Wq[i], Wk[i], Wv[i], Weproj[i]
        xgA = _sc_gather(xn, catA, 128)
        xgB = _sc_gather(xn, catB, 128)
        msgA, eA = _tc_edge_attn(rbf[:EH], xgA[:EH], xgA[EH:], wq, wk, wv,
                                 wep, 0)
        pA, qA = _sc_scatter_add2(msgA, eA, dstA3, z_n, NP, 128, 8)
        msgB, eB = _tc_edge_attn(rbf[EH:], xgB[:EH], xgB[EH:], wq, wk, wv,
                                 wep, EH)
        pB, qB = _sc_scatter_add2(msgB, eB, dstB3, z_n, NP, 128, 8)
        if i < NL - 1:
            gn, bn = ln1_g[i + 1], ln1_b[i + 1]
        else:
            gn, bn = lnf_g, lnf_b
        x, xn = _tc_node_update(x, pA, qA, pB, qB, Wo[i], Wf1[i], Wf2[i],
                                ln2_g[i].reshape(1, D), ln2_b[i].reshape(1, D),
                                gn.reshape(1, D), bn.reshape(1, D))

    no128 = _tc_head(xn, head_w1, head_b1.reshape(1, 512), w2p, b2p)
    pg = _sc_scatter_add(no128, batch3, z_g, GP, 80, 4)
    out = (pg[0, :NG, 0:1] + pg[1, :NG, 0:1]) * (1.0 / math.sqrt(AVG_NUM_NODES))
    return out
